# Initial kernel scaffold; baseline (speedup 1.0000x reference)
import jax
import jax.numpy as jnp
from jax.experimental import pallas as pl


def kernel(node_features, edge_features, W1e, b1e, W2e, b2e, ge, be, W1n, b1n, W2n, b2n, gn, bn, senders, receivers):
    raise NotImplementedError("write your pallas kernel here")


# speedup vs baseline: 3.5183x; 3.5183x over previous
import functools

import jax
import jax.numpy as jnp
from jax import lax
from jax.experimental import pallas as pl
from jax.experimental.pallas import tpu as pltpu
from jax.experimental.pallas import tpu_sc as plsc

N = 10000
E = 320000
D = 128
H = 128

NC = 2
NS = 16
NW = NC * NS
CHUNK = 128
NCHUNK = E // CHUNK

BN = 2000
BE = 2000

_MESH = plsc.VectorSubcoreMesh(
    core_axis_name="c", subcore_axis_name="s", num_cores=NC, num_subcores=NS)


def _pre_body(node_ref, w_ref, ps_ref, pr_ref):
    x = node_ref[...]
    ps_ref[...] = jnp.dot(x, w_ref[0:D, :], preferred_element_type=jnp.float32)
    pr_ref[...] = jnp.dot(x, w_ref[D:2 * D, :], preferred_element_type=jnp.float32)


def _pre_call(node, w_sr):
    return pl.pallas_call(
        _pre_body,
        grid=(N // BN,),
        in_specs=[
            pl.BlockSpec((BN, D), lambda i: (i, 0)),
            pl.BlockSpec((2 * D, H), lambda i: (0, 0)),
        ],
        out_specs=[
            pl.BlockSpec((BN, H), lambda i: (i, 0)),
            pl.BlockSpec((BN, H), lambda i: (i, 0)),
        ],
        out_shape=[
            jax.ShapeDtypeStruct((N, H), jnp.float32),
            jax.ShapeDtypeStruct((N, H), jnp.float32),
        ],
    )(node, w_sr)


@functools.partial(
    pl.kernel,
    out_type=(
        jax.ShapeDtypeStruct((E, H), jnp.float32),
        jax.ShapeDtypeStruct((E, H), jnp.float32),
    ),
    mesh=_MESH,
    scratch_types=[
        pltpu.VMEM((CHUNK,), jnp.int32),
        pltpu.VMEM((CHUNK,), jnp.int32),
        pltpu.VMEM((CHUNK, H), jnp.float32),
        pltpu.VMEM((CHUNK, H), jnp.float32),
        pltpu.SemaphoreType.DMA,
        pltpu.SemaphoreType.DMA,
    ],
)
def _sc_gather(ps_hbm, pr_hbm, s_hbm, r_hbm, gs_hbm, gr_hbm,
               idxs, idxr, bufs, bufr, sems, semr):
    wid = lax.axis_index("s") * NC + lax.axis_index("c")
    base = NCHUNK // NW
    rem = NCHUNK % NW
    cnt = base + jnp.where(wid < rem, 1, 0)
    start = wid * base + jnp.minimum(wid, rem)

    def body(i, carry):
        row = (start + i) * CHUNK
        pltpu.sync_copy(s_hbm.at[pl.ds(row, CHUNK)], idxs)
        pltpu.sync_copy(r_hbm.at[pl.ds(row, CHUNK)], idxr)
        cs = pltpu.async_copy(ps_hbm.at[idxs], bufs, sems)
        cr = pltpu.async_copy(pr_hbm.at[idxr], bufr, semr)
        cs.wait()
        cr.wait()
        pltpu.sync_copy(bufs, gs_hbm.at[pl.ds(row, CHUNK)])
        pltpu.sync_copy(bufr, gr_hbm.at[pl.ds(row, CHUNK)])
        return carry

    lax.fori_loop(0, cnt, body, 0)


def _edge_body(gs_ref, gr_ref, e_ref, w1x_ref, b1_ref, w2_ref, b2_ref,
               g_ref, b_ref, pre_ref, out_ref):
    e = e_ref[...]
    h1 = (gs_ref[...] + gr_ref[...]
          + jnp.dot(e, w1x_ref[...], preferred_element_type=jnp.float32)
          + b1_ref[...])
    h1 = jnp.maximum(h1, 0.0)
    h2 = jnp.dot(h1, w2_ref[...], preferred_element_type=jnp.float32) + b2_ref[...]
    mu = jnp.mean(h2, axis=-1, keepdims=True)
    var = jnp.mean((h2 - mu) ** 2, axis=-1, keepdims=True)
    y = (h2 - mu) / jnp.sqrt(var + 1e-5) * g_ref[...] + b_ref[...]
    pre_ref[...] = y
    out_ref[...] = y + e


def _edge_call(gs, gr, edge, w1x, b1, w2, b2, g, b):
    full = lambda i: (0, 0)
    blk = lambda i: (i, 0)
    return pl.pallas_call(
        _edge_body,
        grid=(E // BE,),
        in_specs=[
            pl.BlockSpec((BE, H), blk),
            pl.BlockSpec((BE, H), blk),
            pl.BlockSpec((BE, D), blk),
            pl.BlockSpec((D, H), full),
            pl.BlockSpec((1, H), full),
            pl.BlockSpec((H, D), full),
            pl.BlockSpec((1, D), full),
            pl.BlockSpec((1, D), full),
            pl.BlockSpec((1, D), full),
        ],
        out_specs=[
            pl.BlockSpec((BE, D), blk),
            pl.BlockSpec((BE, D), blk),
        ],
        out_shape=[
            jax.ShapeDtypeStruct((E, D), jnp.float32),
            jax.ShapeDtypeStruct((E, D), jnp.float32),
        ],
    )(gs, gr, edge, w1x, b1, w2, b2, g, b)


@functools.partial(
    pl.kernel,
    out_type=jax.ShapeDtypeStruct((NC, N, D), jnp.float32),
    mesh=_MESH,
    scratch_types=[
        pltpu.VMEM((CHUNK,), jnp.int32),
        pltpu.VMEM((CHUNK, D), jnp.float32),
        pltpu.VMEM_SHARED((N, D), jnp.float32),
    ],
)
def _sc_scatter(pre_hbm, r_hbm, zeros_hbm, out_hbm, idxr, buf, agg):
    cid = lax.axis_index("c")
    sid = lax.axis_index("s")
    wid = sid * NC + cid
    rows_per = 624
    pltpu.sync_copy(zeros_hbm.at[pl.ds(sid * rows_per, rows_per)],
                    agg.at[pl.ds(sid * rows_per, rows_per)])

    @pl.when(sid == 0)
    def _():
        pltpu.sync_copy(zeros_hbm.at[pl.ds(NS * rows_per, N - NS * rows_per)],
                        agg.at[pl.ds(NS * rows_per, N - NS * rows_per)])

    plsc.subcore_barrier()

    base = NCHUNK // NW
    rem = NCHUNK % NW
    cnt = base + jnp.where(wid < rem, 1, 0)
    start = wid * base + jnp.minimum(wid, rem)

    def body(i, carry):
        row = (start + i) * CHUNK
        pltpu.sync_copy(r_hbm.at[pl.ds(row, CHUNK)], idxr)
        pltpu.sync_copy(pre_hbm.at[pl.ds(row, CHUNK)], buf)
        pltpu.sync_copy(buf, agg.at[idxr], add=True)
        return carry

    lax.fori_loop(0, cnt, body, 0)
    plsc.subcore_barrier()

    @pl.when(sid == 0)
    def _():
        pltpu.sync_copy(agg, out_hbm.at[cid])


def _node_body(node_ref, agg_ref, w1_ref, b1_ref, w2_ref, b2_ref,
               g_ref, b_ref, out_ref):
    x = node_ref[...]
    a = agg_ref[0] + agg_ref[1]
    h1 = (jnp.dot(x, w1_ref[0:D, :], preferred_element_type=jnp.float32)
          + jnp.dot(a, w1_ref[D:2 * D, :], preferred_element_type=jnp.float32)
          + b1_ref[...])
    h1 = jnp.maximum(h1, 0.0)
    h2 = jnp.dot(h1, w2_ref[...], preferred_element_type=jnp.float32) + b2_ref[...]
    mu = jnp.mean(h2, axis=-1, keepdims=True)
    var = jnp.mean((h2 - mu) ** 2, axis=-1, keepdims=True)
    y = (h2 - mu) / jnp.sqrt(var + 1e-5) * g_ref[...] + b_ref[...]
    out_ref[...] = y + x


def _node_call(node, aggs, w1, b1, w2, b2, g, b):
    full = lambda i: (0, 0)
    full3 = lambda i: (0, 0, 0)
    blk = lambda i: (i, 0)
    return pl.pallas_call(
        _node_body,
        grid=(N // BN,),
        in_specs=[
            pl.BlockSpec((BN, D), blk),
            pl.BlockSpec((NC, BN, D), lambda i: (0, i, 0)),
            pl.BlockSpec((2 * D, H), full),
            pl.BlockSpec((1, H), full),
            pl.BlockSpec((H, D), full),
            pl.BlockSpec((1, D), full),
            pl.BlockSpec((1, D), full),
            pl.BlockSpec((1, D), full),
        ],
        out_specs=pl.BlockSpec((BN, D), blk),
        out_shape=jax.ShapeDtypeStruct((N, D), jnp.float32),
    )(node, aggs, w1, b1, w2, b2, g, b)


def kernel(node_features, edge_features, W1e, b1e, W2e, b2e, ge, be,
           W1n, b1n, W2n, b2n, gn, bn, senders, receivers):
    ps, pr = _pre_call(node_features, W1e[0:2 * D])
    gs, gr = _sc_gather(ps, pr, senders, receivers)
    pre, new_edge = _edge_call(
        gs, gr, edge_features, W1e[2 * D:],
        b1e.reshape(1, H), W2e, b2e.reshape(1, D),
        ge.reshape(1, D), be.reshape(1, D))
    zeros = jnp.zeros((N, D), jnp.float32)
    aggs = _sc_scatter(pre, receivers, zeros)
    new_node = _node_call(
        node_features, aggs, W1n, b1n.reshape(1, H), W2n,
        b2n.reshape(1, D), gn.reshape(1, D), bn.reshape(1, D))
    return new_node, new_edge


# speedup vs baseline: 5.2749x; 1.4993x over previous
import functools

import jax
import jax.numpy as jnp
from jax import lax
from jax.experimental import pallas as pl
from jax.experimental.pallas import tpu as pltpu
from jax.experimental.pallas import tpu_sc as plsc

N = 10000
E = 320000
D = 128
H = 128

NC = 2
NS = 16
NW = NC * NS
CHUNK = 128
NCHUNK = E // CHUNK

BN = 2000
BE = 2000

_MESH = plsc.VectorSubcoreMesh(
    core_axis_name="c", subcore_axis_name="s", num_cores=NC, num_subcores=NS)


def _pre_body(node_ref, w_ref, ps_ref, pr_ref):
    x = node_ref[...]
    ps_ref[...] = jnp.dot(x, w_ref[0:D, :], preferred_element_type=jnp.float32)
    pr_ref[...] = jnp.dot(x, w_ref[D:2 * D, :], preferred_element_type=jnp.float32)


def _pre_call(node, w_sr):
    return pl.pallas_call(
        _pre_body,
        grid=(N // BN,),
        in_specs=[
            pl.BlockSpec((BN, D), lambda i: (i, 0)),
            pl.BlockSpec((2 * D, H), lambda i: (0, 0)),
        ],
        out_specs=[
            pl.BlockSpec((BN, H), lambda i: (i, 0)),
            pl.BlockSpec((BN, H), lambda i: (i, 0)),
        ],
        out_shape=[
            jax.ShapeDtypeStruct((N, H), jnp.float32),
            jax.ShapeDtypeStruct((N, H), jnp.float32),
        ],
    )(node, w_sr)


GCPT = 80
GID = GCPT * CHUNK


@functools.partial(
    pl.kernel,
    out_type=jax.ShapeDtypeStruct((E, H), jnp.float32),
    mesh=_MESH,
    scratch_types=[
        pltpu.VMEM((GID,), jnp.int32),
        pltpu.VMEM((GID,), jnp.int32),
        pltpu.VMEM((CHUNK, H), jnp.float32),
        pltpu.VMEM((CHUNK, H), jnp.float32),
        pltpu.VMEM((CHUNK, H), jnp.float32),
        pltpu.VMEM((CHUNK, H), jnp.float32),
        pltpu.VMEM((CHUNK, H), jnp.float32),
        pltpu.VMEM((CHUNK, H), jnp.float32),
        pltpu.SemaphoreType.DMA,
        pltpu.SemaphoreType.DMA,
        pltpu.SemaphoreType.DMA,
        pltpu.SemaphoreType.DMA,
    ],
)
def _sc_gather(ps_hbm, pr_hbm, s_hbm, r_hbm, g_hbm,
               idxs, idxr, bufa0, bufb0, out0, bufa1, bufb1, out1,
               gsem0, gsem1, wsem0, wsem1):
    wid = lax.axis_index("s") * NC + lax.axis_index("c")
    start = (wid * (NCHUNK - GCPT)) // (NW - 1)
    pltpu.sync_copy(s_hbm.at[pl.ds(start * CHUNK, GID)], idxs)
    pltpu.sync_copy(r_hbm.at[pl.ds(start * CHUNK, GID)], idxr)

    bufa = (bufa0, bufa1)
    bufb = (bufb0, bufb1)
    outb = (out0, out1)
    gsem = (gsem0, gsem1)
    wsem = (wsem0, wsem1)

    def fire(i, b):
        pltpu.async_copy(ps_hbm.at[idxs.at[pl.ds(i * CHUNK, CHUNK)]],
                         bufa[b], gsem[b])
        pltpu.async_copy(pr_hbm.at[idxr.at[pl.ds(i * CHUNK, CHUNK)]],
                         bufb[b], gsem[b])

    def wait_gather(i, b):
        pltpu.make_async_copy(ps_hbm.at[idxs.at[pl.ds(i * CHUNK, CHUNK)]],
                              bufa[b], gsem[b]).wait()
        pltpu.make_async_copy(pr_hbm.at[idxr.at[pl.ds(i * CHUNK, CHUNK)]],
                              bufb[b], gsem[b]).wait()

    for b in range(2):
        fire(jnp.int32(b), b)

    def group(g, carry):
        for b in range(2):
            i = g * 2 + b
            wait_gather(i, b)

            @pl.when(g > 0)
            def _():
                pltpu.make_async_copy(
                    outb[b], g_hbm.at[pl.ds(0, CHUNK)], wsem[b]).wait()

            ob, ba, bb = outb[b], bufa[b], bufb[b]

            @plsc.parallel_loop(0, CHUNK, 1, unroll=2)
            def _(r):
                for c in range(H // 16):
                    sl = pl.ds(c * 16, 16)
                    ob[r, sl] = ba[r, sl] + bb[r, sl]

            nxt = i + 2

            @pl.when(nxt < GCPT)
            def _():
                fire(nxt, b)

            pltpu.async_copy(
                ob, g_hbm.at[pl.ds((start + i) * CHUNK, CHUNK)], wsem[b])
        return carry

    lax.fori_loop(0, GCPT // 2, group, 0)
    for b in range(2):
        pltpu.make_async_copy(outb[b], g_hbm.at[pl.ds(0, CHUNK)],
                              wsem[b]).wait()


def _edge_body(gsum_ref, e_ref, w1x_ref, b1_ref, w2_ref, b2_ref,
               g_ref, b_ref, pre_ref, out_ref):
    e = e_ref[...]
    h1 = (gsum_ref[...]
          + jnp.dot(e, w1x_ref[...], preferred_element_type=jnp.float32)
          + b1_ref[...])
    h1 = jnp.maximum(h1, 0.0)
    h2 = jnp.dot(h1, w2_ref[...], preferred_element_type=jnp.float32) + b2_ref[...]
    mu = jnp.mean(h2, axis=-1, keepdims=True)
    var = jnp.mean((h2 - mu) ** 2, axis=-1, keepdims=True)
    y = (h2 - mu) / jnp.sqrt(var + 1e-5) * g_ref[...] + b_ref[...]
    pre_ref[...] = y
    out_ref[...] = y + e


def _edge_call(gsum, edge, w1x, b1, w2, b2, g, b):
    full = lambda i: (0, 0)
    blk = lambda i: (i, 0)
    return pl.pallas_call(
        _edge_body,
        grid=(E // BE,),
        in_specs=[
            pl.BlockSpec((BE, H), blk),
            pl.BlockSpec((BE, D), blk),
            pl.BlockSpec((D, H), full),
            pl.BlockSpec((1, H), full),
            pl.BlockSpec((H, D), full),
            pl.BlockSpec((1, D), full),
            pl.BlockSpec((1, D), full),
            pl.BlockSpec((1, D), full),
        ],
        out_specs=[
            pl.BlockSpec((BE, D), blk),
            pl.BlockSpec((BE, D), blk),
        ],
        out_shape=[
            jax.ShapeDtypeStruct((E, D), jnp.float32),
            jax.ShapeDtypeStruct((E, D), jnp.float32),
        ],
    )(gsum, edge, w1x, b1, w2, b2, g, b)


SCPT = NCHUNK // NW


@functools.partial(
    pl.kernel,
    out_type=jax.ShapeDtypeStruct((NC, N, D), jnp.float32),
    mesh=_MESH,
    scratch_types=[
        pltpu.VMEM((CHUNK,), jnp.int32),
        pltpu.VMEM((CHUNK,), jnp.int32),
        pltpu.VMEM((CHUNK, D), jnp.float32),
        pltpu.VMEM((CHUNK, D), jnp.float32),
        pltpu.VMEM_SHARED((N, D), jnp.float32),
        pltpu.SemaphoreType.DMA,
        pltpu.SemaphoreType.DMA,
    ],
)
def _sc_scatter(pre_hbm, r_hbm, zeros_hbm, out_hbm,
                idx0, idx1, buf0, buf1, agg, lsem0, lsem1):
    cid = lax.axis_index("c")
    sid = lax.axis_index("s")
    wid = sid * NC + cid
    rows_per = 624
    pltpu.sync_copy(zeros_hbm.at[pl.ds(sid * rows_per, rows_per)],
                    agg.at[pl.ds(sid * rows_per, rows_per)])

    @pl.when(sid == 0)
    def _():
        pltpu.sync_copy(zeros_hbm.at[pl.ds(NS * rows_per, N - NS * rows_per)],
                        agg.at[pl.ds(NS * rows_per, N - NS * rows_per)])

    plsc.subcore_barrier()

    start = wid * SCPT
    idx = (idx0, idx1)
    buf = (buf0, buf1)
    lsem = (lsem0, lsem1)

    def fire(i, b):
        row = (start + i) * CHUNK
        pltpu.async_copy(r_hbm.at[pl.ds(row, CHUNK)], idx[b], lsem[b])
        pltpu.async_copy(pre_hbm.at[pl.ds(row, CHUNK)], buf[b], lsem[b])

    def wait_load(b):
        pltpu.make_async_copy(r_hbm.at[pl.ds(0, CHUNK)], idx[b],
                              lsem[b]).wait()
        pltpu.make_async_copy(pre_hbm.at[pl.ds(0, CHUNK)], buf[b],
                              lsem[b]).wait()

    for b in range(2):
        fire(jnp.int32(b), b)

    def group(g, carry):
        for b in range(2):
            i = g * 2 + b
            wait_load(b)
            pltpu.sync_copy(buf[b], agg.at[idx[b]], add=True)
            nxt = i + 2

            @pl.when(nxt < SCPT)
            def _():
                fire(nxt, b)
        return carry

    lax.fori_loop(0, SCPT // 2, group, 0)

    @pl.when(wid < NCHUNK - NW * SCPT)
    def _():
        row = (NW * SCPT + wid) * CHUNK
        pltpu.sync_copy(r_hbm.at[pl.ds(row, CHUNK)], idx0)
        pltpu.sync_copy(pre_hbm.at[pl.ds(row, CHUNK)], buf0)
        pltpu.sync_copy(buf0, agg.at[idx0], add=True)

    plsc.subcore_barrier()

    @pl.when(sid == 0)
    def _():
        pltpu.sync_copy(agg, out_hbm.at[cid])


def _node_body(node_ref, agg_ref, w1_ref, b1_ref, w2_ref, b2_ref,
               g_ref, b_ref, out_ref):
    x = node_ref[...]
    a = agg_ref[0] + agg_ref[1]
    h1 = (jnp.dot(x, w1_ref[0:D, :], preferred_element_type=jnp.float32)
          + jnp.dot(a, w1_ref[D:2 * D, :], preferred_element_type=jnp.float32)
          + b1_ref[...])
    h1 = jnp.maximum(h1, 0.0)
    h2 = jnp.dot(h1, w2_ref[...], preferred_element_type=jnp.float32) + b2_ref[...]
    mu = jnp.mean(h2, axis=-1, keepdims=True)
    var = jnp.mean((h2 - mu) ** 2, axis=-1, keepdims=True)
    y = (h2 - mu) / jnp.sqrt(var + 1e-5) * g_ref[...] + b_ref[...]
    out_ref[...] = y + x


def _node_call(node, aggs, w1, b1, w2, b2, g, b):
    full = lambda i: (0, 0)
    full3 = lambda i: (0, 0, 0)
    blk = lambda i: (i, 0)
    return pl.pallas_call(
        _node_body,
        grid=(N // BN,),
        in_specs=[
            pl.BlockSpec((BN, D), blk),
            pl.BlockSpec((NC, BN, D), lambda i: (0, i, 0)),
            pl.BlockSpec((2 * D, H), full),
            pl.BlockSpec((1, H), full),
            pl.BlockSpec((H, D), full),
            pl.BlockSpec((1, D), full),
            pl.BlockSpec((1, D), full),
            pl.BlockSpec((1, D), full),
        ],
        out_specs=pl.BlockSpec((BN, D), blk),
        out_shape=jax.ShapeDtypeStruct((N, D), jnp.float32),
    )(node, aggs, w1, b1, w2, b2, g, b)


def kernel(node_features, edge_features, W1e, b1e, W2e, b2e, ge, be,
           W1n, b1n, W2n, b2n, gn, bn, senders, receivers):
    ps, pr = _pre_call(node_features, W1e[0:2 * D])
    gsum = _sc_gather(ps, pr, senders, receivers)
    pre, new_edge = _edge_call(
        gsum, edge_features, W1e[2 * D:],
        b1e.reshape(1, H), W2e, b2e.reshape(1, D),
        ge.reshape(1, D), be.reshape(1, D))
    zeros = jnp.zeros((N, D), jnp.float32)
    aggs = _sc_scatter(pre, receivers, zeros)
    new_node = _node_call(
        node_features, aggs, W1n, b1n.reshape(1, H), W2n,
        b2n.reshape(1, D), gn.reshape(1, D), bn.reshape(1, D))
    return new_node, new_edge
